# Initial kernel scaffold; baseline (speedup 1.0000x reference)
import jax
import jax.numpy as jnp
from jax.experimental import pallas as pl


def kernel(x, w1, w2):
    raise NotImplementedError("write your pallas kernel here")


# speedup vs baseline: 1.0788x; 1.0788x over previous
import functools

import jax
import jax.numpy as jnp
from jax.experimental import pallas as pl
from jax.experimental.pallas import tpu as pltpu

_MIB = 1024 * 1024
_VMEM_LIMIT = 48 * _MIB
_BLOCK_BUDGET = 28 * _MIB


def _se_kernel(x_ref, w1_ref, w2_ref, o_ref, *, inv_hw, nb):
    w1 = w1_ref[...]
    w2 = w2_ref[...]
    cols = [
        jnp.sum(jnp.abs(x_ref[i]), axis=-1, keepdims=True) * inv_hw
        for i in range(nb)
    ]
    pooled = cols[0] if nb == 1 else jnp.concatenate(cols, axis=1)
    hidden = jnp.maximum(
        jnp.dot(w1, pooled, preferred_element_type=jnp.float32), 0.0)
    logits = jnp.dot(w2, hidden, preferred_element_type=jnp.float32)
    scale = 1.0 / (1.0 + jnp.exp(-logits))
    for i in range(nb):
        o_ref[i] = x_ref[i] * scale[:, i:i + 1]


def _pick_nb(n, bytes_per_sample):
    nb = max(1, _BLOCK_BUDGET // (4 * bytes_per_sample))
    nb = int(min(n, nb))
    if n >= 2:
        nb = max(1, min(nb, n // 2))
    while n % nb:
        nb -= 1
    return nb


def kernel(x, w1, w2):
    n, c, h, w = x.shape
    hw = h * w
    mid = w1.shape[0]
    x2 = x.reshape(n, c, hw)
    itemsize = jnp.dtype(x.dtype).itemsize
    nb = _pick_nb(n, c * hw * itemsize)
    body = functools.partial(_se_kernel, inv_hw=1.0 / hw, nb=nb)
    out = pl.pallas_call(
        body,
        out_shape=jax.ShapeDtypeStruct((n, c, hw), x.dtype),
        grid=(n // nb,),
        in_specs=[
            pl.BlockSpec((nb, c, hw), lambda i: (i, 0, 0)),
            pl.BlockSpec((mid, c), lambda i: (0, 0)),
            pl.BlockSpec((c, mid), lambda i: (0, 0)),
        ],
        out_specs=pl.BlockSpec((nb, c, hw), lambda i: (i, 0, 0)),
        compiler_params=pltpu.CompilerParams(
            dimension_semantics=("parallel",),
            vmem_limit_bytes=_VMEM_LIMIT,
        ),
    )(x2, w1, w2)
    return out.reshape(n, c, h, w)


# speedup vs baseline: 4.8260x; 4.4734x over previous
import functools

import jax
import jax.numpy as jnp
from jax import lax
from jax.experimental import pallas as pl
from jax.experimental.pallas import tpu as pltpu

_MIB = 1024 * 1024
_VMEM_LIMIT = 58 * _MIB
_BLOCK_BUDGET = 52 * _MIB


def _se_kernel(x_ref, w1_ref, w2_ref, o_ref, *, inv_hw):
    xb = x_ref[...]
    pooled = jnp.sum(jnp.abs(xb), axis=0) * inv_hw
    hidden = jnp.maximum(
        lax.dot_general(pooled, w1_ref[...], (((1,), (1,)), ((), ())),
                        preferred_element_type=jnp.float32), 0.0)
    logits = lax.dot_general(hidden, w2_ref[...], (((1,), (1,)), ((), ())),
                             preferred_element_type=jnp.float32)
    scale = 1.0 / (1.0 + jnp.exp(-logits))
    o_ref[...] = xb * scale[None]


def _pick_nb(n, bytes_per_sample):
    budget = max(1, _BLOCK_BUDGET // (4 * bytes_per_sample))
    divisors = [d for d in range(1, n + 1) if n % d == 0 and d <= budget]
    mult8 = [d for d in divisors if d % 8 == 0]
    return max(mult8) if mult8 else max(divisors)


def kernel(x, w1, w2):
    n, c, h, w = x.shape
    hw = h * w
    mid = w1.shape[0]
    xt = jnp.transpose(x, (2, 3, 0, 1)).reshape(hw, n, c)
    itemsize = jnp.dtype(x.dtype).itemsize
    nb = _pick_nb(n, c * hw * itemsize)
    body = functools.partial(_se_kernel, inv_hw=1.0 / hw)
    out = pl.pallas_call(
        body,
        out_shape=jax.ShapeDtypeStruct((hw, n, c), x.dtype),
        grid=(n // nb,),
        in_specs=[
            pl.BlockSpec((hw, nb, c), lambda i: (0, i, 0)),
            pl.BlockSpec((mid, c), lambda i: (0, 0)),
            pl.BlockSpec((c, mid), lambda i: (0, 0)),
        ],
        out_specs=pl.BlockSpec((hw, nb, c), lambda i: (0, i, 0)),
        compiler_params=pltpu.CompilerParams(
            dimension_semantics=("parallel",),
            vmem_limit_bytes=_VMEM_LIMIT,
        ),
    )(xt, w1, w2)
    return jnp.transpose(out.reshape(h, w, n, c), (2, 3, 0, 1))
